# baseline (speedup 1.0000x reference)
import functools

import jax
import jax.numpy as jnp
from jax import lax
from jax.experimental import pallas as pl
from jax.experimental.pallas import tpu as pltpu
from jax.experimental.pallas import tpu_sc as plsc

_NC = 2
_NS = 16
_CH = 2048


def _dense_body(n, f0, f1, x_ref, p_ref, nrm_ref, wz_ref, uz_ref, bz_ref,
                wr_ref, ur_ref, br_ref, wh_ref, uh_ref, bh_ref, w0_ref,
                lw_ref, hv_ref, w_scr):
    t = pl.program_id(0)

    @pl.when(t == 0)
    def _init():
        w_scr[...] = w0_ref[...]

    x = x_ref[0]
    p = p_ref[...]
    y = lax.dot_general(p, x, (((1,), (1,)), ((), ())),
                        preferred_element_type=jnp.float32)
    y = y / nrm_ref[...]

    iota = lax.broadcasted_iota(jnp.int32, (1, n), 1)
    row_k = lax.broadcasted_iota(jnp.int32, (f1, 1), 0)
    y = jnp.where(iota < n, y, -jnp.inf)

    def step(i, carry):
        y_cur, idxs, ms = carry
        m = jnp.max(y_cur)
        idx = jnp.min(jnp.where(y_cur == m, iota, n))
        sel = row_k == i
        idxs = jnp.where(sel, idx, idxs)
        ms = jnp.where(sel, m, ms)
        y_cur = jnp.where(iota == idx, -jnp.inf, y_cur)
        return y_cur, idxs, ms

    _, idxs, ms = lax.fori_loop(
        0, f1, step,
        (y, jnp.zeros((f1, 1), jnp.int32), jnp.zeros((f1, 1), jnp.float32)))

    col = lax.broadcasted_iota(jnp.int32, (f1, n), 1)
    st = jnp.where(col == idxs, ms, 0.0)
    zs = jnp.dot(st, x, preferred_element_type=jnp.float32, precision=lax.Precision.HIGHEST)

    w = w_scr[...]
    zg = jax.nn.sigmoid(jnp.dot(zs, wz_ref[...], preferred_element_type=jnp.float32)
                        + jnp.dot(w, uz_ref[...], preferred_element_type=jnp.float32)
                        + bz_ref[...])
    rg = jax.nn.sigmoid(jnp.dot(zs, wr_ref[...], preferred_element_type=jnp.float32)
                        + jnp.dot(w, ur_ref[...], preferred_element_type=jnp.float32)
                        + br_ref[...])
    ht = jnp.tanh(jnp.dot(zs, wh_ref[...], preferred_element_type=jnp.float32)
                  + jnp.dot(rg * w, uh_ref[...], preferred_element_type=jnp.float32)
                  + bh_ref[...])
    w_new = (1.0 - zg) * w + zg * ht
    w_scr[...] = w_new

    v = jnp.dot(lw_ref[...], w_new, preferred_element_type=jnp.float32, precision=lax.Precision.HIGHEST)
    hv = lax.dot_general(v, x, (((1,), (1,)), ((), ())),
                         preferred_element_type=jnp.float32, precision=lax.Precision.HIGHEST)
    hv_ref[0] = hv


def _dense_stage(X, W_init, p, W_Z, U_Z, B_Z, W_R, U_R, B_R, W_H, U_H, B_H,
                 lin_w):
    T, n, f0 = X.shape
    f1 = W_init.shape[1]
    full = lambda s: pl.BlockSpec(s, lambda t: (0,) * len(s))
    hv = pl.pallas_call(
        functools.partial(_dense_body, n, f0, f1),
        grid=(T,),
        in_specs=[
            pl.BlockSpec((1, n, f0), lambda t: (t, 0, 0)),
            full((1, f0)), full((1, 1)),
            full((f0, f0)), full((f0, f0)), full((f1, f0)),
            full((f0, f0)), full((f0, f0)), full((f1, f0)),
            full((f0, f0)), full((f0, f0)), full((f1, f0)),
            full((f1, f0)), full((1, f1)),
        ],
        out_specs=pl.BlockSpec((1, 1, n), lambda t: (t, 0, 0)),
        out_shape=jax.ShapeDtypeStruct((T, 1, n), jnp.float32),
        scratch_shapes=[pltpu.VMEM((f1, f0), jnp.float32)],
        compiler_params=pltpu.CompilerParams(
            dimension_semantics=("arbitrary",)),
    )(X, p.reshape(1, f0), jnp.linalg.norm(p, 2).reshape(1, 1),
      W_Z.T, U_Z.T, B_Z.T, W_R.T, U_R.T, B_R.T,
      W_H.T, U_H.T, B_H.T, W_init.T, lin_w)
    return hv.reshape(T, n)


def _sc_body(t_per_core, npad, ept, hv_hbm, cols_hbm, vals_hbm, rows_hbm,
             linb_hbm, out_hbm, hv_v, cols_v, vals_v, contrib_v, rows_v,
             init_v, bias_v, acc_sh):
    cid = lax.axis_index("c")
    sid = lax.axis_index("s")
    nslice = npad // _NS
    nchunk = ept // _CH

    pltpu.sync_copy(linb_hbm, bias_v)
    b = bias_v[...]

    def fill(j, _):
        init_v[pl.ds(j * 16, 16)] = b
        return 0

    lax.fori_loop(0, nslice // 16, fill, 0)

    def per_t(ti, _):
        t = cid * t_per_core + ti
        pltpu.sync_copy(init_v, acc_sh.at[pl.ds(sid * nslice, nslice)])
        pltpu.sync_copy(hv_hbm.at[t], hv_v)
        plsc.subcore_barrier()
        ebase = sid * ept
        rbase = sid * (ept // 128)

        def per_chunk(ci, _):
            off = ebase + ci * _CH
            pltpu.sync_copy(cols_hbm.at[t, pl.ds(off, _CH)], cols_v)
            pltpu.sync_copy(vals_hbm.at[t, pl.ds(off, _CH)], vals_v)
            pltpu.sync_copy(rows_hbm.at[t, pl.ds(rbase + ci * 16, 16)],
                            rows_v)

            def grp(g, _):
                s = g * 16
                idx = cols_v[pl.ds(s, 16)]
                gath = plsc.load_gather(hv_v, [idx])
                contrib_v[pl.ds(s, 16)] = gath * vals_v[pl.ds(s, 16)]
                return 0

            lax.fori_loop(0, _CH // 16, grp, 0)
            for j in range(1):
                pltpu.sync_copy(contrib_v.at[pl.ds(j * 128, 128)],
                                acc_sh.at[rows_v.at[j]], add=True)
            return 0

        lax.fori_loop(0, nchunk, per_chunk, 0)
        plsc.subcore_barrier()
        pltpu.sync_copy(acc_sh.at[pl.ds(sid * nslice, nslice)],
                        out_hbm.at[t, pl.ds(sid * nslice, nslice)])
        plsc.subcore_barrier()
        return 0

    lax.fori_loop(0, t_per_core, per_t, 0)


def _sparse_stage(hv, edge_vals, edge_rows, edge_cols, lin_b):
    T, n = hv.shape
    e = edge_vals.shape[1]
    npad = ((n + (_NS * 16) - 1) // (_NS * 16)) * (_NS * 16)
    ept = -(-e // (_NS * _CH)) * _CH
    ep = ept * _NS
    pad = ep - e
    cols_p = jnp.pad(edge_cols, ((0, 0), (0, pad)))
    vals_p = jnp.pad(edge_vals, ((0, 0), (0, pad)))
    rows_p = jnp.pad(edge_rows, ((0, 0), (0, pad)), constant_values=n)
    rows_p = rows_p.reshape(T, ep // 128, 128)
    linb16 = jnp.full((16,), lin_b[0], jnp.float32)

    mesh = plsc.VectorSubcoreMesh(core_axis_name="c", subcore_axis_name="s")
    out = pl.kernel(
        functools.partial(_sc_body, T // _NC, npad, ept),
        out_type=jax.ShapeDtypeStruct((T, npad), jnp.float32),
        mesh=mesh,
        compiler_params=pltpu.CompilerParams(needs_layout_passes=False),
        scratch_types=[
            pltpu.VMEM((n,), jnp.float32),
            pltpu.VMEM((_CH,), jnp.int32),
            pltpu.VMEM((_CH,), jnp.float32),
            pltpu.VMEM((_CH,), jnp.float32),
            pltpu.VMEM((16, 128), jnp.int32),
            pltpu.VMEM((npad // _NS,), jnp.float32),
            pltpu.VMEM((16,), jnp.float32),
            pltpu.VMEM_SHARED((npad,), jnp.float32),
        ],
    )(hv, cols_p, vals_p, rows_p, linb16)
    return out[:, :n]


def kernel(X, W_init, edge_vals, p, W_Z, U_Z, B_Z, W_R, U_R, B_R, W_H, U_H,
           B_H, lin_w, lin_b, edge_rows, edge_cols):
    hv = _dense_stage(X, W_init, p, W_Z, U_Z, B_Z, W_R, U_R, B_R,
                      W_H, U_H, B_H, lin_w)
    return _sparse_stage(hv, edge_vals, edge_rows, edge_cols, lin_b)


# speedup vs baseline: 1.0291x; 1.0291x over previous
import functools

import jax
import jax.numpy as jnp
from jax import lax
from jax.experimental import pallas as pl
from jax.experimental.pallas import tpu as pltpu
from jax.experimental.pallas import tpu_sc as plsc

_NC = 2
_NS = 16
_CH = 2048


def _dense_body(n, f0, f1, x_ref, p_ref, nrm_ref, wz_ref, uz_ref, bz_ref,
                wr_ref, ur_ref, br_ref, wh_ref, uh_ref, bh_ref, w0_ref,
                lw_ref, hv_ref, w_scr):
    t = pl.program_id(0)

    @pl.when(t == 0)
    def _init():
        w_scr[...] = w0_ref[...]

    x = x_ref[0]
    p = p_ref[...]
    y = lax.dot_general(p, x, (((1,), (1,)), ((), ())),
                        preferred_element_type=jnp.float32)
    y = y / nrm_ref[...]

    iota = lax.broadcasted_iota(jnp.int32, (1, n), 1)
    row_k = lax.broadcasted_iota(jnp.int32, (f1, 1), 0)
    y = jnp.where(iota < n, y, -jnp.inf)

    def step(i, carry):
        y_cur, idxs, ms = carry
        m = jnp.max(y_cur)
        idx = jnp.min(jnp.where(y_cur == m, iota, n))
        sel = row_k == i
        idxs = jnp.where(sel, idx, idxs)
        ms = jnp.where(sel, m, ms)
        y_cur = jnp.where(iota == idx, -jnp.inf, y_cur)
        return y_cur, idxs, ms

    _, idxs, ms = lax.fori_loop(
        0, f1, step,
        (y, jnp.zeros((f1, 1), jnp.int32), jnp.zeros((f1, 1), jnp.float32)))

    col = lax.broadcasted_iota(jnp.int32, (f1, n), 1)
    st = jnp.where(col == idxs, ms, 0.0)
    zs = jnp.dot(st, x, preferred_element_type=jnp.float32, precision=lax.Precision.HIGHEST)

    w = w_scr[...]
    zg = jax.nn.sigmoid(jnp.dot(zs, wz_ref[...], preferred_element_type=jnp.float32)
                        + jnp.dot(w, uz_ref[...], preferred_element_type=jnp.float32)
                        + bz_ref[...])
    rg = jax.nn.sigmoid(jnp.dot(zs, wr_ref[...], preferred_element_type=jnp.float32)
                        + jnp.dot(w, ur_ref[...], preferred_element_type=jnp.float32)
                        + br_ref[...])
    ht = jnp.tanh(jnp.dot(zs, wh_ref[...], preferred_element_type=jnp.float32)
                  + jnp.dot(rg * w, uh_ref[...], preferred_element_type=jnp.float32)
                  + bh_ref[...])
    w_new = (1.0 - zg) * w + zg * ht
    w_scr[...] = w_new

    v = jnp.dot(lw_ref[...], w_new, preferred_element_type=jnp.float32, precision=lax.Precision.HIGHEST)
    hv = lax.dot_general(v, x, (((1,), (1,)), ((), ())),
                         preferred_element_type=jnp.float32, precision=lax.Precision.HIGHEST)
    hv_ref[0] = hv


def _dense_stage(X, W_init, p, W_Z, U_Z, B_Z, W_R, U_R, B_R, W_H, U_H, B_H,
                 lin_w):
    T, n, f0 = X.shape
    f1 = W_init.shape[1]
    full = lambda s: pl.BlockSpec(s, lambda t: (0,) * len(s))
    hv = pl.pallas_call(
        functools.partial(_dense_body, n, f0, f1),
        grid=(T,),
        in_specs=[
            pl.BlockSpec((1, n, f0), lambda t: (t, 0, 0)),
            full((1, f0)), full((1, 1)),
            full((f0, f0)), full((f0, f0)), full((f1, f0)),
            full((f0, f0)), full((f0, f0)), full((f1, f0)),
            full((f0, f0)), full((f0, f0)), full((f1, f0)),
            full((f1, f0)), full((1, f1)),
        ],
        out_specs=pl.BlockSpec((1, 1, n), lambda t: (t, 0, 0)),
        out_shape=jax.ShapeDtypeStruct((T, 1, n), jnp.float32),
        scratch_shapes=[pltpu.VMEM((f1, f0), jnp.float32)],
        compiler_params=pltpu.CompilerParams(
            dimension_semantics=("arbitrary",)),
    )(X, p.reshape(1, f0), jnp.linalg.norm(p, 2).reshape(1, 1),
      W_Z.T, U_Z.T, B_Z.T, W_R.T, U_R.T, B_R.T,
      W_H.T, U_H.T, B_H.T, W_init.T, lin_w)
    return hv.reshape(T, n)


def _sc_body(t_per_core, npad, ept, hv_hbm, cols_hbm, vals_hbm, rows_hbm,
             linb_hbm, out_hbm, hv_v, cols_v, vals_v, contrib_v, rows_v,
             init_v, bias_v, acc_sh):
    cid = lax.axis_index("c")
    sid = lax.axis_index("s")
    nslice = npad // _NS
    nchunk = ept // _CH

    pltpu.sync_copy(linb_hbm, bias_v)
    b = bias_v[...]

    def fill(j, _):
        init_v[pl.ds(j * 16, 16)] = b
        return 0

    lax.fori_loop(0, nslice // 16, fill, 0)

    def per_t(ti, _):
        t = cid * t_per_core + ti
        pltpu.sync_copy(init_v, acc_sh.at[pl.ds(sid * nslice, nslice)])
        pltpu.sync_copy(hv_hbm.at[t], hv_v)
        plsc.subcore_barrier()
        ebase = sid * ept
        rbase = sid * (ept // 128)

        def per_chunk(ci, _):
            off = ebase + ci * _CH
            pltpu.sync_copy(cols_hbm.at[t, pl.ds(off, _CH)], cols_v)
            pltpu.sync_copy(vals_hbm.at[t, pl.ds(off, _CH)], vals_v)
            pltpu.sync_copy(rows_hbm.at[t, pl.ds(rbase + ci * 16, 16)],
                            rows_v)

            def grp(g, _):
                s = g * 16
                contrib_v[pl.ds(s, 16)] = vals_v[pl.ds(s, 16)]
                return 0

            lax.fori_loop(0, _CH // 16, grp, 0)
            for j in range(1):
                pltpu.sync_copy(contrib_v.at[pl.ds(j * 128, 128)],
                                acc_sh.at[rows_v.at[j]], add=True)
            return 0

        lax.fori_loop(0, nchunk, per_chunk, 0)
        plsc.subcore_barrier()
        pltpu.sync_copy(acc_sh.at[pl.ds(sid * nslice, nslice)],
                        out_hbm.at[t, pl.ds(sid * nslice, nslice)])
        plsc.subcore_barrier()
        return 0

    lax.fori_loop(0, t_per_core, per_t, 0)


def _sparse_stage(hv, edge_vals, edge_rows, edge_cols, lin_b):
    T, n = hv.shape
    e = edge_vals.shape[1]
    npad = ((n + (_NS * 16) - 1) // (_NS * 16)) * (_NS * 16)
    ept = -(-e // (_NS * _CH)) * _CH
    ep = ept * _NS
    pad = ep - e
    cols_p = jnp.pad(edge_cols, ((0, 0), (0, pad)))
    vals_p = jnp.pad(edge_vals, ((0, 0), (0, pad)))
    rows_p = jnp.pad(edge_rows, ((0, 0), (0, pad)), constant_values=n)
    rows_p = rows_p.reshape(T, ep // 128, 128)
    linb16 = jnp.full((16,), lin_b[0], jnp.float32)

    mesh = plsc.VectorSubcoreMesh(core_axis_name="c", subcore_axis_name="s")
    out = pl.kernel(
        functools.partial(_sc_body, T // _NC, npad, ept),
        out_type=jax.ShapeDtypeStruct((T, npad), jnp.float32),
        mesh=mesh,
        compiler_params=pltpu.CompilerParams(needs_layout_passes=False),
        scratch_types=[
            pltpu.VMEM((n,), jnp.float32),
            pltpu.VMEM((_CH,), jnp.int32),
            pltpu.VMEM((_CH,), jnp.float32),
            pltpu.VMEM((_CH,), jnp.float32),
            pltpu.VMEM((16, 128), jnp.int32),
            pltpu.VMEM((npad // _NS,), jnp.float32),
            pltpu.VMEM((16,), jnp.float32),
            pltpu.VMEM_SHARED((npad,), jnp.float32),
        ],
    )(hv, cols_p, vals_p, rows_p, linb16)
    return out[:, :n]


def kernel(X, W_init, edge_vals, p, W_Z, U_Z, B_Z, W_R, U_R, B_R, W_H, U_H,
           B_H, lin_w, lin_b, edge_rows, edge_cols):
    hv = _dense_stage(X, W_init, p, W_Z, U_Z, B_Z, W_R, U_R, B_R,
                      W_H, U_H, B_H, lin_w)
    return _sparse_stage(hv, edge_vals, edge_rows, edge_cols, lin_b)


# speedup vs baseline: 1.1187x; 1.0870x over previous
import functools

import jax
import jax.numpy as jnp
from jax import lax
from jax.experimental import pallas as pl
from jax.experimental.pallas import tpu as pltpu
from jax.experimental.pallas import tpu_sc as plsc

_NC = 2
_NS = 16
_CH = 2048


def _dense_body(n, f0, f1, x_ref, p_ref, nrm_ref, wz_ref, uz_ref, bz_ref,
                wr_ref, ur_ref, br_ref, wh_ref, uh_ref, bh_ref, w0_ref,
                lw_ref, hv_ref, w_scr):
    t = pl.program_id(0)

    @pl.when(t == 0)
    def _init():
        w_scr[...] = w0_ref[...]

    x = x_ref[0]
    p = p_ref[...]
    y = lax.dot_general(p, x, (((1,), (1,)), ((), ())),
                        preferred_element_type=jnp.float32)
    y = y / nrm_ref[...]

    iota = lax.broadcasted_iota(jnp.int32, (1, n), 1)
    row_k = lax.broadcasted_iota(jnp.int32, (f1, 1), 0)
    y = jnp.where(iota < n, y, -jnp.inf)

    def step(i, carry):
        y_cur, idxs, ms = carry
        m = jnp.max(y_cur)
        idx = jnp.min(jnp.where(y_cur == m, iota, n))
        sel = row_k == i
        idxs = jnp.where(sel, idx, idxs)
        ms = jnp.where(sel, m, ms)
        y_cur = jnp.where(iota == idx, -jnp.inf, y_cur)
        return y_cur, idxs, ms

    _, idxs, ms = lax.fori_loop(
        0, f1, step,
        (y, jnp.zeros((f1, 1), jnp.int32), jnp.zeros((f1, 1), jnp.float32)))

    col = lax.broadcasted_iota(jnp.int32, (f1, n), 1)
    st = jnp.where(col == idxs, ms, 0.0)
    zs = jnp.dot(st, x, preferred_element_type=jnp.float32, precision=lax.Precision.HIGHEST)

    w = w_scr[...]
    zg = jax.nn.sigmoid(jnp.dot(zs, wz_ref[...], preferred_element_type=jnp.float32)
                        + jnp.dot(w, uz_ref[...], preferred_element_type=jnp.float32)
                        + bz_ref[...])
    rg = jax.nn.sigmoid(jnp.dot(zs, wr_ref[...], preferred_element_type=jnp.float32)
                        + jnp.dot(w, ur_ref[...], preferred_element_type=jnp.float32)
                        + br_ref[...])
    ht = jnp.tanh(jnp.dot(zs, wh_ref[...], preferred_element_type=jnp.float32)
                  + jnp.dot(rg * w, uh_ref[...], preferred_element_type=jnp.float32)
                  + bh_ref[...])
    w_new = (1.0 - zg) * w + zg * ht
    w_scr[...] = w_new

    v = jnp.dot(lw_ref[...], w_new, preferred_element_type=jnp.float32, precision=lax.Precision.HIGHEST)
    hv = lax.dot_general(v, x, (((1,), (1,)), ((), ())),
                         preferred_element_type=jnp.float32, precision=lax.Precision.HIGHEST)
    hv_ref[0] = hv


def _dense_stage(X, W_init, p, W_Z, U_Z, B_Z, W_R, U_R, B_R, W_H, U_H, B_H,
                 lin_w):
    T, n, f0 = X.shape
    f1 = W_init.shape[1]
    full = lambda s: pl.BlockSpec(s, lambda t: (0,) * len(s))
    hv = pl.pallas_call(
        functools.partial(_dense_body, n, f0, f1),
        grid=(T,),
        in_specs=[
            pl.BlockSpec((1, n, f0), lambda t: (t, 0, 0)),
            full((1, f0)), full((1, 1)),
            full((f0, f0)), full((f0, f0)), full((f1, f0)),
            full((f0, f0)), full((f0, f0)), full((f1, f0)),
            full((f0, f0)), full((f0, f0)), full((f1, f0)),
            full((f1, f0)), full((1, f1)),
        ],
        out_specs=pl.BlockSpec((1, 1, n), lambda t: (t, 0, 0)),
        out_shape=jax.ShapeDtypeStruct((T, 1, n), jnp.float32),
        scratch_shapes=[pltpu.VMEM((f1, f0), jnp.float32)],
        compiler_params=pltpu.CompilerParams(
            dimension_semantics=("arbitrary",)),
    )(X, p.reshape(1, f0), jnp.linalg.norm(p, 2).reshape(1, 1),
      W_Z.T, U_Z.T, B_Z.T, W_R.T, U_R.T, B_R.T,
      W_H.T, U_H.T, B_H.T, W_init.T, lin_w)
    return hv.reshape(T, n)


def _sc_body(t_per_core, npad, nrow_total, maxng, hv_hbm, cols_hbm, vals_hbm,
             rows_hbm, linb_hbm, out_hbm, hv_v, cols_v, vals_v, rows_v,
             init_v, bias_v, acc_sh, sem):
    cid = lax.axis_index("c")
    sid = lax.axis_index("s")
    r8 = nrow_total // 8
    off = pl.multiple_of(8 * (sid * r8 // _NS), 8)
    off1 = jnp.where(sid == _NS - 1, nrow_total, 8 * ((sid + 1) * r8 // _NS))
    ng = off1 - off
    wbase = npad // _NS
    woff = sid * wbase

    pltpu.sync_copy(linb_hbm, bias_v)
    b = bias_v[...]

    def fill(j, _):
        init_v[pl.ds(j * 16, 16)] = b
        return 0

    lax.fori_loop(0, wbase // 16, fill, 0)

    def init_slice():
        pltpu.sync_copy(init_v, acc_sh.at[pl.ds(woff, wbase)])

    def write_slice(t):
        pltpu.sync_copy(acc_sh.at[pl.ds(woff, wbase)],
                        out_hbm.at[t, pl.ds(woff, wbase)])

    init_slice()
    KW = 16

    def per_t(ti, _):
        t = cid * t_per_core + ti
        pltpu.sync_copy(hv_hbm.at[t], hv_v)
        eoff = pl.multiple_of(off * 128, 128)
        pltpu.sync_copy(cols_hbm.at[t, pl.ds(eoff, maxng * 128)], cols_v)
        pltpu.sync_copy(vals_hbm.at[t, pl.ds(eoff, maxng * 128)], vals_v)
        pltpu.sync_copy(rows_hbm.at[t, pl.ds(off, maxng)], rows_v)
        plsc.subcore_barrier()

        nb8 = ng // 8

        def blk(b, _):
            rblk = rows_v.at[pl.ds(pl.multiple_of(b * 8, 8), 8)]
            for u in range(8):
                sbase = pl.multiple_of((b * 8 + u) * 128, 128)
                for v in range(8):
                    sl = pl.ds(sbase + v * 16, 16)
                    idx = cols_v[sl]
                    vals_v[sl] = plsc.load_gather(hv_v, [idx]) * vals_v[sl]
                pltpu.async_copy(vals_v.at[pl.ds(sbase, 128)],
                                 acc_sh.at[rblk.at[u]], sem, add=True)

            @pl.when(b >= 2)
            def _():
                pblk = rows_v.at[pl.ds(pl.multiple_of((b - 2) * 8, 8), 8)]
                for u in range(8):
                    sbase = pl.multiple_of(((b - 2) * 8 + u) * 128, 128)
                    pltpu.make_async_copy(vals_v.at[pl.ds(sbase, 128)],
                                          acc_sh.at[pblk.at[u]], sem).wait()
            return 0

        lax.fori_loop(0, nb8, blk, 0)

        def drain(j, _):
            dblk = rows_v.at[pl.ds(pl.multiple_of(j * 8, 8), 8)]
            for u in range(8):
                sbase = pl.multiple_of((j * 8 + u) * 128, 128)
                pltpu.make_async_copy(vals_v.at[pl.ds(sbase, 128)],
                                      acc_sh.at[dblk.at[u]], sem).wait()
            return 0

        lax.fori_loop(nb8 - 2, nb8, drain, 0)
        plsc.subcore_barrier()
        write_slice(t)
        init_slice()
        return 0

    lax.fori_loop(0, t_per_core, per_t, 0)


def _sparse_stage(hv, edge_vals, edge_rows, edge_cols, lin_b):
    T, n = hv.shape
    e = edge_vals.shape[1]
    nrow_total = -(-e // 1024) * 8
    ep = nrow_total * 128
    if ep != e:
        edge_cols = jnp.pad(edge_cols, ((0, 0), (0, ep - e)))
        edge_vals = jnp.pad(edge_vals, ((0, 0), (0, ep - e)))
        edge_rows = jnp.pad(edge_rows, ((0, 0), (0, ep - e)))
    r8 = nrow_total // 8
    offs = [8 * (s * r8 // _NS) for s in range(_NS)] + [nrow_total]
    maxng = max(offs[s + 1] - offs[s] for s in range(_NS))
    assert offs[_NS - 1] + maxng <= nrow_total and maxng % 8 == 0
    rows_r = edge_rows.reshape(T, nrow_total, 128)
    linb16 = jnp.full((16,), lin_b[0], jnp.float32)

    npad = -(-n // (_NS * 128)) * _NS * 128
    mesh = plsc.VectorSubcoreMesh(core_axis_name="c", subcore_axis_name="s")
    out = pl.kernel(
        functools.partial(_sc_body, T // _NC, npad, nrow_total, maxng),
        out_type=jax.ShapeDtypeStruct((T, npad), jnp.float32),
        mesh=mesh,
        compiler_params=pltpu.CompilerParams(needs_layout_passes=False),
        scratch_types=[
            pltpu.VMEM((n,), jnp.float32),
            pltpu.VMEM((maxng * 128,), jnp.int32),
            pltpu.VMEM((maxng * 128,), jnp.float32),
            pltpu.VMEM((maxng, 128), jnp.int32),
            pltpu.VMEM((npad // _NS,), jnp.float32),
            pltpu.VMEM((16,), jnp.float32),
            pltpu.VMEM_SHARED((npad,), jnp.float32),
            pltpu.SemaphoreType.DMA,
        ],
    )(hv, edge_cols, edge_vals, rows_r, linb16)
    return out[:, :n]


def kernel(X, W_init, edge_vals, p, W_Z, U_Z, B_Z, W_R, U_R, B_R, W_H, U_H,
           B_H, lin_w, lin_b, edge_rows, edge_cols):
    hv = _dense_stage(X, W_init, p, W_Z, U_Z, B_Z, W_R, U_R, B_R,
                      W_H, U_H, B_H, lin_w)
    return _sparse_stage(hv, edge_vals, edge_rows, edge_cols, lin_b)
